# baseline (speedup 1.0000x reference)
import functools

import jax
import jax.numpy as jnp
from jax import lax
from jax.experimental import pallas as pl
from jax.experimental.pallas import tpu as pltpu
from jax.experimental.pallas import tpu_sc as plsc

NC, NS, L = 2, 16, 16
NW = NC * NS

BATCH = 4096
SEQ = 200
EMBED_DIM = 64
TOTAL = BATCH * SEQ
PER_W = TOTAL // NW
BLK = 128
K = 4
SUPER = K * BLK
NBLK = PER_W // BLK
NSUP = PER_W // SUPER
GROUPS = NSUP // 2
ENC_ROWS = SEQ + BLK - 8


def _pos_encoding(seq_len, d):
    position = jnp.arange(0, seq_len, dtype=jnp.float32)[:, None]
    div_term = jnp.exp(jnp.arange(0, d, 2, dtype=jnp.float32) * -(jnp.log(10000.0) / d))
    enc = jnp.zeros((seq_len, d), dtype=jnp.float32)
    enc = enc.at[:, 0::2].set(jnp.sin(position * div_term))
    enc = enc.at[:, 1::2].set(jnp.cos(position * div_term[: d // 2]))
    return enc


def _body(weight_hbm, idx_hbm, enc_hbm, out_hbm, idx_all, enc_v,
          rows0, rows1, g0, g1, s0, s1):
    rows_bufs = [rows0, rows1]
    gsems = [g0, g1]
    ssems = [s0, s1]
    wid = lax.axis_index("s") * NC + lax.axis_index("c")
    base = wid * PER_W

    pltpu.sync_copy(idx_hbm.at[pl.ds(wid * NSUP, NSUP), :], idx_all)
    pltpu.sync_copy(enc_hbm, enc_v)

    def start_gather(c, b):
        pltpu.async_copy(
            weight_hbm.at[idx_all.at[c]], rows_bufs[b], gsems[b]
        )

    def wait_gather(c, b):
        pltpu.make_async_copy(
            weight_hbm.at[idx_all.at[c]], rows_bufs[b], gsems[b]
        ).wait()

    def start_store(c, b):
        pltpu.async_copy(
            rows_bufs[b], out_hbm.at[pl.ds(base + c * SUPER, SUPER)], ssems[b]
        )

    def wait_store(c, b):
        pltpu.make_async_copy(
            rows_bufs[b], out_hbm.at[pl.ds(base + c * SUPER, SUPER)], ssems[b]
        ).wait()

    start_gather(0, 0)

    def group_body(g, carry):
        for b in range(2):
            c = g * 2 + b
            ob = 1 - b
            wait_gather(c, b)
            if b == 0:

                @pl.when(g > 0)
                def _():
                    wait_store(c - 1, ob)

                start_gather(c + 1, ob)
            else:
                wait_store(c - 1, ob)

                @pl.when(g < GROUPS - 1)
                def _():
                    start_gather(c + 1, ob)

            rv = rows_bufs[b]
            for s in range(K):
                off = ((c * K + s) * BLK) % SEQ

                @plsc.parallel_loop(0, BLK, unroll=8)
                def _(r):
                    e = off + r
                    for j in range(EMBED_DIM // L):
                        sl = pl.ds(j * L, L)
                        row = s * BLK + r
                        rv[row, sl] = rv[row, sl] + enc_v[e, sl]

            start_store(c, b)
        return carry

    lax.fori_loop(0, GROUPS, group_body, 0)
    wait_store(NSUP - 1, 1)


@jax.jit
def _embed(text, weight, enc_ext):
    idx2d = text.reshape(TOTAL // SUPER, SUPER).astype(jnp.int32)
    mesh = plsc.VectorSubcoreMesh(
        core_axis_name="c", subcore_axis_name="s", num_cores=NC, num_subcores=NS
    )
    out = pl.kernel(
        _body,
        out_type=jax.ShapeDtypeStruct((TOTAL, EMBED_DIM), jnp.float32),
        mesh=mesh,
        scratch_types=[
            pltpu.VMEM((NSUP, SUPER), jnp.int32),
            pltpu.VMEM((ENC_ROWS, EMBED_DIM), jnp.float32),
            pltpu.VMEM((SUPER, EMBED_DIM), jnp.float32),
            pltpu.VMEM((SUPER, EMBED_DIM), jnp.float32),
            pltpu.SemaphoreType.DMA,
            pltpu.SemaphoreType.DMA,
            pltpu.SemaphoreType.DMA,
            pltpu.SemaphoreType.DMA,
        ],
        compiler_params=pltpu.CompilerParams(use_tc_tiling_on_sc=False),
    )(weight, idx2d, enc_ext)
    return out.reshape(BATCH, SEQ, EMBED_DIM)


def kernel(text, weight):
    enc = _pos_encoding(SEQ, EMBED_DIM)
    enc_ext = jnp.concatenate([enc, enc[: ENC_ROWS - SEQ]], axis=0)
    return _embed(text, weight, enc_ext)
